# Initial kernel scaffold; baseline (speedup 1.0000x reference)
import functools

import numpy as np
import jax
import jax.numpy as jnp
from jax import lax
from jax.experimental import pallas as pl
from jax.experimental.pallas import tpu as pltpu
from jax.experimental.pallas import tpu_sc as plsc

_T = 524288
_TMASK = _T - 1
_NLEV = 16
_NL = (16, 22, 30, 42, 58, 80, 111, 153, 212, 294, 406, 561, 776, 1072,
       1482, 2048)
_PI2 = jnp.int32(np.uint32(2654435761).view(np.int32))
_PI3 = jnp.int32(805459861)
_B = 262144

_NC, _NS = 2, 16
_NW = _NC * _NS
_PPW = _B // _NW
_C = 128
_NCHUNK = _PPW // _C
_NSUB = _C // 16
_NHALF = _C // 8


def _sc_hash_features(xT, xdup, tbl):
  mesh = plsc.VectorSubcoreMesh(core_axis_name="c", subcore_axis_name="s")

  @functools.partial(
      pl.kernel,
      out_type=jax.ShapeDtypeStruct((2 * _NLEV, _B), jnp.float32),
      mesh=mesh,
      scratch_types=[
          pltpu.VMEM((3, _C), jnp.float32),
          pltpu.VMEM((3, 2 * _C), jnp.float32),
          pltpu.VMEM((_NLEV, _NSUB, 128), jnp.int32),
          pltpu.VMEM((_NLEV, _NSUB, 128, 2), jnp.float32),
          pltpu.VMEM((2 * _NLEV, _C), jnp.float32),
          pltpu.SemaphoreType.DMA((_NLEV,)),
      ],
  )
  def body(xT_hbm, xd_hbm, tbl_hbm, out_hbm, xv, xdv, idxv, rowsv, featv,
           sems):
    wid = lax.axis_index("s") * _NC + lax.axis_index("c")
    base0 = wid * _PPW
    lanes = lax.iota(jnp.int32, 16)
    fpat = lanes & 1
    colpat = lax.shift_right_logical(lanes, 1)
    rowsflat = rowsv.reshape(_NLEV * _NSUB * 256)

    def chunk_body(ci, _):
      base = base0 + ci * _C
      pltpu.sync_copy(xT_hbm.at[:, pl.ds(base, _C)], xv)
      pltpu.sync_copy(xd_hbm.at[:, pl.ds(2 * base, 2 * _C)], xdv)

      for l in range(_NLEV):
        n = float(_NL[l])

        def hash_body(p, _, l=l, n=n):
          xq = xv[0, pl.ds(p * 16, 16)]
          yq = xv[1, pl.ds(p * 16, 16)]
          zq = xv[2, pl.ds(p * 16, 16)]
          px = (xq / 3.0 + 0.5) * n
          py = (yq / 3.0 + 0.5) * n
          pz = (zq / 3.0 + 0.5) * n
          x0 = px.astype(jnp.int32)
          y0 = py.astype(jnp.int32)
          z0 = pz.astype(jnp.int32)
          ax0 = x0
          ax1 = x0 + 1
          by0 = y0 * _PI2
          by1 = by0 + _PI2
          bz0 = z0 * _PI3
          bz1 = bz0 + _PI3
          t00 = by0 ^ bz0
          t10 = by1 ^ bz0
          t01 = by0 ^ bz1
          t11 = by1 ^ bz1
          loff = l * _T
          ts = (t00, t10, t01, t11)
          for j in range(8):
            ax = ax1 if (j & 1) else ax0
            h = ((ax ^ ts[j >> 1]) & _TMASK) + loff
            idxv[l, p, pl.ds(j * 16, 16)] = h
          return 0

        lax.fori_loop(0, _NSUB, hash_body, 0, unroll=True)
        pltpu.async_copy(tbl_hbm.at[idxv.at[l]], rowsv.at[l], sems.at[l])

      for l in range(_NLEV):
        n = float(_NL[l])
        pltpu.make_async_copy(tbl_hbm.at[idxv.at[l]], rowsv.at[l],
                              sems.at[l]).wait()
        rowpat = fpat + 2 * l

        def interp_body(hp, _, l=l, n=n, rowpat=rowpat):
          xq = xdv[0, pl.ds(hp * 16, 16)]
          yq = xdv[1, pl.ds(hp * 16, 16)]
          zq = xdv[2, pl.ds(hp * 16, 16)]
          px = (xq / 3.0 + 0.5) * n
          py = (yq / 3.0 + 0.5) * n
          pz = (zq / 3.0 + 0.5) * n
          lx = px - px.astype(jnp.int32).astype(jnp.float32)
          ly = py - py.astype(jnp.int32).astype(jnp.float32)
          lz = pz - pz.astype(jnp.int32).astype(jnp.float32)
          p = lax.shift_right_logical(hp, 1)
          h = hp & 1
          fbase = (l * _NSUB + p) * 256 + h * 16
          cf = [rowsflat[pl.ds(fbase + j * 32, 16)] for j in range(8)]
          cx0 = cf[0] + lx * (cf[1] - cf[0])
          cx1 = cf[2] + lx * (cf[3] - cf[2])
          cx2 = cf[4] + lx * (cf[5] - cf[4])
          cx3 = cf[6] + lx * (cf[7] - cf[6])
          cy0 = cx0 + ly * (cx1 - cx0)
          cy1 = cx2 + ly * (cx3 - cx2)
          acc = cy0 + lz * (cy1 - cy0)
          plsc.store_scatter(featv, [rowpat, colpat + hp * 8], acc)
          return 0

        lax.fori_loop(0, _NHALF, interp_body, 0, unroll=True)

      pltpu.sync_copy(featv, out_hbm.at[:, pl.ds(base, _C)])
      return 0

    lax.fori_loop(0, _NCHUNK, chunk_body, 0)

  return body(xT, xdup, tbl)


_BT = 2048


def _tc_mlp_body(f_ref, d_ref, x_ref, w1, b1, w2, b2, cw1h, cw1d, cb1, cw2,
                 cb2, cw3, cb3, rgb_ref, sig_ref):
  hi = jax.lax.Precision.HIGHEST
  fb = f_ref[...]
  h1 = jnp.maximum(
      jnp.dot(w1[...], fb, precision=hi, preferred_element_type=jnp.float32)
      + b1[...], 0.0)
  h = jnp.dot(w2[...], h1, precision=hi,
              preferred_element_type=jnp.float32) + b2[...]
  db = d_ref[...]
  parts = [db]
  for j in range(4):
    s = float(2.0 ** j)
    parts.append(jnp.sin(s * db))
    parts.append(jnp.cos(s * db))
  denc = jnp.concatenate(parts, axis=0)
  r1 = jnp.maximum(
      jnp.dot(cw1h[...], h, precision=hi, preferred_element_type=jnp.float32)
      + jnp.dot(cw1d[...], denc, precision=hi,
                preferred_element_type=jnp.float32) + cb1[...], 0.0)
  r2 = jnp.maximum(
      jnp.dot(cw2[...], r1, precision=hi,
              preferred_element_type=jnp.float32) + cb2[...], 0.0)
  rgb_full = jax.nn.sigmoid(
      jnp.dot(cw3[...], r2, precision=hi,
              preferred_element_type=jnp.float32) + cb3[...])
  xb = x_ref[...]
  xn = xb / 3.0
  m = ((jnp.abs(xn[0:1, :]) < 0.5) & (jnp.abs(xn[1:2, :]) < 0.5)
       & (jnp.abs(xn[2:3, :]) < 0.5))
  sig_ref[...] = jnp.exp(jnp.where(m, h[0:1, :], -100000.0))
  rgb_ref[...] = jnp.where(m, rgb_full, 0.0)


def _tc_mlp(featsT, dT, xT, w1, b1, w2, b2, cw1h, cw1d, cb1, cw2, cb2, cw3,
            cb3):
  grid = (_B // _BT,)

  def batch_spec(rows):
    return pl.BlockSpec((rows, _BT), lambda i: (0, i))

  def full_spec(shape):
    return pl.BlockSpec(shape, lambda i: (0, 0))

  return pl.pallas_call(
      _tc_mlp_body,
      grid=grid,
      in_specs=[
          batch_spec(2 * _NLEV),
          batch_spec(3),
          batch_spec(3),
          full_spec(w1.shape), full_spec(b1.shape),
          full_spec(w2.shape), full_spec(b2.shape),
          full_spec(cw1h.shape), full_spec(cw1d.shape), full_spec(cb1.shape),
          full_spec(cw2.shape), full_spec(cb2.shape),
          full_spec(cw3.shape), full_spec(cb3.shape),
      ],
      out_specs=[batch_spec(3), batch_spec(1)],
      out_shape=[
          jax.ShapeDtypeStruct((3, _B), jnp.float32),
          jax.ShapeDtypeStruct((1, _B), jnp.float32),
      ],
  )(featsT, dT, xT, w1, b1, w2, b2, cw1h, cw1d, cb1, cw2, cb2, cw3, cb3)


def kernel(x, d, tables, d_w1, d_b1, d_w2, d_b2, c_w1, c_b1, c_w2, c_b2,
           c_w3, c_b3):
  x = x.astype(jnp.float32)
  d = d.astype(jnp.float32)
  xT = x.T
  dT = d.T
  xdup = jnp.repeat(xT, 2, axis=1)
  tbl = tables.reshape(_NLEV * _T, 2)

  featsT = _sc_hash_features(xT, xdup, tbl)

  rgbT, sigT = _tc_mlp(
      featsT, dT, xT,
      d_w1, d_b1.reshape(-1, 1),
      d_w2, d_b2.reshape(-1, 1),
      c_w1[:, :16], c_w1[:, 16:], c_b1.reshape(-1, 1),
      c_w2, c_b2.reshape(-1, 1),
      c_w3, c_b3.reshape(-1, 1),
  )
  return rgbT.T, sigT.reshape(_B)


# speedup vs baseline: 20.8555x; 20.8555x over previous
import functools

import numpy as np
import jax
import jax.numpy as jnp
from jax import lax
from jax.experimental import pallas as pl
from jax.experimental.pallas import tpu as pltpu
from jax.experimental.pallas import tpu_sc as plsc

_T = 524288
_TMASK = _T - 1
_NLEV = 16
_NL = (16, 22, 30, 42, 58, 80, 111, 153, 212, 294, 406, 561, 776, 1072,
       1482, 2048)
_PI2 = np.uint32(2654435761).view(np.int32)
_PI3 = np.int32(805459861)
_B = 262144

_NC, _NS = 2, 16
_NW = _NC * _NS
_PPW = _B // _NW
_C = 32
_NCHUNK = _PPW // _C
_NSUB = _C // 16
_NHALF = _C // 8


def _sc_hash_features(xT, xdup, tblp):
  mesh = plsc.VectorSubcoreMesh(core_axis_name="c", subcore_axis_name="s")

  @functools.partial(
      pl.kernel,
      out_type=jax.ShapeDtypeStruct((2 * _NLEV, _B), jnp.float32),
      mesh=mesh,
      scratch_types=[
          pltpu.VMEM((3, _C), jnp.float32),
          pltpu.VMEM((3, 2 * _C), jnp.float32),
          pltpu.VMEM((_NLEV, _NSUB, 128), jnp.int32),
          pltpu.VMEM((_NLEV, _NSUB, 128), jnp.int32),
          pltpu.VMEM((_NLEV, _NSUB, 128, 16), jnp.float32),
          pltpu.VMEM((2 * _NLEV, _C), jnp.float32),
          pltpu.SemaphoreType.DMA((_NLEV,)),
      ],
      compiler_params=pltpu.CompilerParams(needs_layout_passes=False,
                                           use_tc_tiling_on_sc=False),
  )
  def body(xT_hbm, xd_hbm, tbl_hbm, out_hbm, xv, xdv, idxv, lanev, rowsv,
           featv, sems):
    wid = lax.axis_index("s") * _NC + lax.axis_index("c")
    base0 = wid * _PPW
    lanes = lax.iota(jnp.int32, 16)
    fpat = lanes & 1
    colpat = lax.shift_right_logical(lanes, 1)

    def chunk_body(ci, _):
      base = base0 + ci * _C
      pltpu.sync_copy(xT_hbm.at[:, pl.ds(base, _C)], xv)
      pltpu.sync_copy(xd_hbm.at[:, pl.ds(2 * base, 2 * _C)], xdv)

      for l in range(_NLEV):
        n = float(_NL[l])

        def hash_body(p, _, l=l, n=n):
          xq = xv[0, pl.ds(p * 16, 16)]
          yq = xv[1, pl.ds(p * 16, 16)]
          zq = xv[2, pl.ds(p * 16, 16)]
          px = (xq / 3.0 + 0.5) * n
          py = (yq / 3.0 + 0.5) * n
          pz = (zq / 3.0 + 0.5) * n
          x0 = px.astype(jnp.int32)
          y0 = py.astype(jnp.int32)
          z0 = pz.astype(jnp.int32)
          ax0 = x0
          ax1 = x0 + 1
          by0 = y0 * _PI2
          by1 = by0 + _PI2
          bz0 = z0 * _PI3
          bz1 = bz0 + _PI3
          t00 = by0 ^ bz0
          t10 = by1 ^ bz0
          t01 = by0 ^ bz1
          t11 = by1 ^ bz1
          loff = l * (_T // 8)
          ts = (t00, t10, t01, t11)
          for j in range(8):
            ax = ax1 if (j & 1) else ax0
            h = (ax ^ ts[j >> 1]) & _TMASK
            idxv[l, p, pl.ds(j * 16, 16)] = lax.shift_right_logical(h, 3) + loff
            lanev[l, p, pl.ds(j * 16, 16)] = (h & 7) * 2
          return 0

        lax.fori_loop(0, _NSUB, hash_body, 0)

        def fire_body(g, _, l=l):
          pltpu.async_copy(tbl_hbm.at[idxv.at[l, g]], rowsv.at[l, g],
                           sems.at[l])
          return 0

        lax.fori_loop(0, _NSUB, fire_body, 0)

      for l in range(_NLEV):
        n = float(_NL[l])
        def drain_body(g, _, l=l):
          pltpu.make_async_copy(tbl_hbm.at[idxv.at[l, g]], rowsv.at[l, g],
                                sems.at[l]).wait()
          return 0

        lax.fori_loop(0, _NSUB, drain_body, 0)
        rowpat = fpat + 2 * l

        def interp_body(hp, _, l=l, n=n, rowpat=rowpat):
          xq = xdv[0, pl.ds(hp * 16, 16)]
          yq = xdv[1, pl.ds(hp * 16, 16)]
          zq = xdv[2, pl.ds(hp * 16, 16)]
          px = (xq / 3.0 + 0.5) * n
          py = (yq / 3.0 + 0.5) * n
          pz = (zq / 3.0 + 0.5) * n
          lx = px - px.astype(jnp.int32).astype(jnp.float32)
          ly = py - py.astype(jnp.int32).astype(jnp.float32)
          lz = pz - pz.astype(jnp.int32).astype(jnp.float32)
          p = lax.shift_right_logical(hp, 1)
          h = hp & 1
          lvec = jnp.full((16,), l, jnp.int32)
          pvec = jnp.full((16,), p, jnp.int32)
          rvec = colpat + (h * 8)
          cf = []
          for j in range(8):
            rj = rvec + j * 16
            lj = plsc.load_gather(lanev, [lvec, pvec, rj]) + fpat
            cf.append(plsc.load_gather(rowsv, [lvec, pvec, rj, lj]))
          cx0 = cf[0] + lx * (cf[1] - cf[0])
          cx1 = cf[2] + lx * (cf[3] - cf[2])
          cx2 = cf[4] + lx * (cf[5] - cf[4])
          cx3 = cf[6] + lx * (cf[7] - cf[6])
          cy0 = cx0 + ly * (cx1 - cx0)
          cy1 = cx2 + ly * (cx3 - cx2)
          acc = cy0 + lz * (cy1 - cy0)
          plsc.store_scatter(featv, [rowpat, colpat + hp * 8], acc)
          return 0

        lax.fori_loop(0, _NHALF, interp_body, 0)

      pltpu.sync_copy(featv, out_hbm.at[:, pl.ds(base, _C)])
      return 0

    lax.fori_loop(0, _NCHUNK, chunk_body, 0)

  return body(xT, xdup, tblp)


_BT = 2048


def _tc_mlp_body(f_ref, d_ref, x_ref, w1, b1, w2, b2, cw1h, cw1d, cb1, cw2,
                 cb2, cw3, cb3, rgb_ref, sig_ref):
  hi = jax.lax.Precision.HIGHEST
  fb = f_ref[...]
  h1 = jnp.maximum(
      jnp.dot(w1[...], fb, precision=hi, preferred_element_type=jnp.float32)
      + b1[...], 0.0)
  h = jnp.dot(w2[...], h1, precision=hi,
              preferred_element_type=jnp.float32) + b2[...]
  db = d_ref[...]
  parts = [db]
  for j in range(4):
    s = float(2.0 ** j)
    parts.append(jnp.sin(s * db))
    parts.append(jnp.cos(s * db))
  denc = jnp.concatenate(parts, axis=0)
  r1 = jnp.maximum(
      jnp.dot(cw1h[...], h, precision=hi, preferred_element_type=jnp.float32)
      + jnp.dot(cw1d[...], denc, precision=hi,
                preferred_element_type=jnp.float32) + cb1[...], 0.0)
  r2 = jnp.maximum(
      jnp.dot(cw2[...], r1, precision=hi,
              preferred_element_type=jnp.float32) + cb2[...], 0.0)
  rgb_full = jax.nn.sigmoid(
      jnp.dot(cw3[...], r2, precision=hi,
              preferred_element_type=jnp.float32) + cb3[...])
  xb = x_ref[...]
  xn = xb / 3.0
  m = ((jnp.abs(xn[0:1, :]) < 0.5) & (jnp.abs(xn[1:2, :]) < 0.5)
       & (jnp.abs(xn[2:3, :]) < 0.5))
  sig_ref[...] = jnp.exp(jnp.where(m, h[0:1, :], -100000.0))
  rgb_ref[...] = jnp.where(m, rgb_full, 0.0)


def _tc_mlp(featsT, dT, xT, w1, b1, w2, b2, cw1h, cw1d, cb1, cw2, cb2, cw3,
            cb3):
  grid = (_B // _BT,)

  def batch_spec(rows):
    return pl.BlockSpec((rows, _BT), lambda i: (0, i))

  def full_spec(shape):
    return pl.BlockSpec(shape, lambda i: (0, 0))

  return pl.pallas_call(
      _tc_mlp_body,
      grid=grid,
      in_specs=[
          batch_spec(2 * _NLEV),
          batch_spec(3),
          batch_spec(3),
          full_spec(w1.shape), full_spec(b1.shape),
          full_spec(w2.shape), full_spec(b2.shape),
          full_spec(cw1h.shape), full_spec(cw1d.shape), full_spec(cb1.shape),
          full_spec(cw2.shape), full_spec(cb2.shape),
          full_spec(cw3.shape), full_spec(cb3.shape),
      ],
      out_specs=[batch_spec(3), batch_spec(1)],
      out_shape=[
          jax.ShapeDtypeStruct((3, _B), jnp.float32),
          jax.ShapeDtypeStruct((1, _B), jnp.float32),
      ],
  )(featsT, dT, xT, w1, b1, w2, b2, cw1h, cw1d, cb1, cw2, cb2, cw3, cb3)


def kernel(x, d, tables, d_w1, d_b1, d_w2, d_b2, c_w1, c_b1, c_w2, c_b2,
           c_w3, c_b3):
  x = x.astype(jnp.float32)
  d = d.astype(jnp.float32)
  xT = x.T
  dT = d.T
  xdup = jnp.repeat(xT, 2, axis=1)
  tblp = tables.reshape(_NLEV * _T // 8, 16)

  featsT = _sc_hash_features(xT, xdup, tblp)

  rgbT, sigT = _tc_mlp(
      featsT, dT, xT,
      d_w1, d_b1.reshape(-1, 1),
      d_w2, d_b2.reshape(-1, 1),
      c_w1[:, :16], c_w1[:, 16:], c_b1.reshape(-1, 1),
      c_w2, c_b2.reshape(-1, 1),
      c_w3, c_b3.reshape(-1, 1),
  )
  return rgbT.T, sigT.reshape(_B)


# speedup vs baseline: 21.6932x; 1.0402x over previous
import functools

import numpy as np
import jax
import jax.numpy as jnp
from jax import lax
from jax.experimental import pallas as pl
from jax.experimental.pallas import tpu as pltpu
from jax.experimental.pallas import tpu_sc as plsc

_T = 524288
_TMASK = _T - 1
_NLEV = 16
_NL = (16, 22, 30, 42, 58, 80, 111, 153, 212, 294, 406, 561, 776, 1072,
       1482, 2048)
_PI2 = np.uint32(2654435761).view(np.int32)
_PI3 = np.int32(805459861)
_B = 262144

_NC, _NS = 2, 16
_NW = _NC * _NS
_PPW = _B // _NW
_C = 32
_NCHUNK = _PPW // _C
_NSUB = _C // 16
_NHALF = _C // 8


def _sc_hash_features(xT, tblp):
  mesh = plsc.VectorSubcoreMesh(core_axis_name="c", subcore_axis_name="s")

  @functools.partial(
      pl.kernel,
      out_type=jax.ShapeDtypeStruct((2 * _NLEV, _B), jnp.float32),
      mesh=mesh,
      scratch_types=[
          pltpu.VMEM((3, _C), jnp.float32),
          pltpu.VMEM((_NLEV, _NSUB, 128), jnp.int32),
          pltpu.VMEM((_NLEV, _NSUB, 128), jnp.int32),
          pltpu.VMEM((_NLEV, _NSUB, 128, 16), jnp.float32),
          pltpu.VMEM((2 * _NLEV, _C), jnp.float32),
          pltpu.SemaphoreType.DMA((_NLEV,)),
      ],
      compiler_params=pltpu.CompilerParams(needs_layout_passes=False,
                                           use_tc_tiling_on_sc=False),
  )
  def body(xT_hbm, tbl_hbm, out_hbm, xv, idxv, lanev, rowsv,
           featv, sems):
    wid = lax.axis_index("s") * _NC + lax.axis_index("c")
    base0 = wid * _PPW
    lanes = lax.iota(jnp.int32, 16)
    fpat = lanes & 1
    colpat = lax.shift_right_logical(lanes, 1)

    def chunk_body(ci, _):
      base = base0 + ci * _C
      pltpu.sync_copy(xT_hbm.at[:, pl.ds(base, _C)], xv)

      for l in range(_NLEV):
        n = float(_NL[l])

        def hash_body(p, _, l=l, n=n):
          xq = xv[0, pl.ds(p * 16, 16)]
          yq = xv[1, pl.ds(p * 16, 16)]
          zq = xv[2, pl.ds(p * 16, 16)]
          px = (xq / 3.0 + 0.5) * n
          py = (yq / 3.0 + 0.5) * n
          pz = (zq / 3.0 + 0.5) * n
          x0 = px.astype(jnp.int32)
          y0 = py.astype(jnp.int32)
          z0 = pz.astype(jnp.int32)
          ax0 = x0
          ax1 = x0 + 1
          by0 = y0 * _PI2
          by1 = by0 + _PI2
          bz0 = z0 * _PI3
          bz1 = bz0 + _PI3
          t00 = by0 ^ bz0
          t10 = by1 ^ bz0
          t01 = by0 ^ bz1
          t11 = by1 ^ bz1
          loff = l * (_T // 8)
          ts = (t00, t10, t01, t11)
          for j in range(8):
            ax = ax1 if (j & 1) else ax0
            h = (ax ^ ts[j >> 1]) & _TMASK
            idxv[l, p, pl.ds(j * 16, 16)] = lax.shift_right_logical(h, 3) + loff
            lanev[l, p, pl.ds(j * 16, 16)] = (h & 7) * 2
          return 0

        lax.fori_loop(0, _NSUB, hash_body, 0)

        def fire_body(g, _, l=l):
          pltpu.async_copy(tbl_hbm.at[idxv.at[l, g]], rowsv.at[l, g],
                           sems.at[l])
          return 0

        lax.fori_loop(0, _NSUB, fire_body, 0)

      for l in range(_NLEV):
        n = float(_NL[l])
        def drain_body(g, _, l=l):
          pltpu.make_async_copy(tbl_hbm.at[idxv.at[l, g]], rowsv.at[l, g],
                                sems.at[l]).wait()
          return 0

        lax.fori_loop(0, _NSUB, drain_body, 0)
        rowpat = fpat + 2 * l

        def interp_body(hp, _, l=l, n=n, rowpat=rowpat):
          cvec = colpat + hp * 8
          xq = plsc.load_gather(xv, [jnp.zeros((16,), jnp.int32), cvec])
          yq = plsc.load_gather(xv, [jnp.ones((16,), jnp.int32), cvec])
          zq = plsc.load_gather(xv, [jnp.full((16,), 2, jnp.int32), cvec])
          px = (xq / 3.0 + 0.5) * n
          py = (yq / 3.0 + 0.5) * n
          pz = (zq / 3.0 + 0.5) * n
          lx = px - px.astype(jnp.int32).astype(jnp.float32)
          ly = py - py.astype(jnp.int32).astype(jnp.float32)
          lz = pz - pz.astype(jnp.int32).astype(jnp.float32)
          p = lax.shift_right_logical(hp, 1)
          h = hp & 1
          lvec = jnp.full((16,), l, jnp.int32)
          pvec = jnp.full((16,), p, jnp.int32)
          rvec = colpat + (h * 8)
          cf = []
          for j in range(8):
            rj = rvec + j * 16
            lj = plsc.load_gather(lanev, [lvec, pvec, rj]) + fpat
            cf.append(plsc.load_gather(rowsv, [lvec, pvec, rj, lj]))
          cx0 = cf[0] + lx * (cf[1] - cf[0])
          cx1 = cf[2] + lx * (cf[3] - cf[2])
          cx2 = cf[4] + lx * (cf[5] - cf[4])
          cx3 = cf[6] + lx * (cf[7] - cf[6])
          cy0 = cx0 + ly * (cx1 - cx0)
          cy1 = cx2 + ly * (cx3 - cx2)
          acc = cy0 + lz * (cy1 - cy0)
          plsc.store_scatter(featv, [rowpat, colpat + hp * 8], acc)
          return 0

        lax.fori_loop(0, _NHALF, interp_body, 0)

      pltpu.sync_copy(featv, out_hbm.at[:, pl.ds(base, _C)])
      return 0

    lax.fori_loop(0, _NCHUNK, chunk_body, 0)

  return body(xT, tblp)


_RPK = _NLEV * _T // 8

_BT = 2048


def _tc_mlp_body(f_ref, d_ref, x_ref, w1, b1, w2, b2, cw1h, cw1d, cb1, cw2,
                 cb2, cw3, cb3, rgb_ref, sig_ref):
  hi = jax.lax.Precision.HIGHEST
  fb = f_ref[...]
  h1 = jnp.maximum(
      jnp.dot(w1[...], fb, precision=hi, preferred_element_type=jnp.float32)
      + b1[...], 0.0)
  h = jnp.dot(w2[...], h1, precision=hi,
              preferred_element_type=jnp.float32) + b2[...]
  db = d_ref[...]
  parts = [db]
  for j in range(4):
    s = float(2.0 ** j)
    parts.append(jnp.sin(s * db))
    parts.append(jnp.cos(s * db))
  denc = jnp.concatenate(parts, axis=0)
  r1 = jnp.maximum(
      jnp.dot(cw1h[...], h, precision=hi, preferred_element_type=jnp.float32)
      + jnp.dot(cw1d[...], denc, precision=hi,
                preferred_element_type=jnp.float32) + cb1[...], 0.0)
  r2 = jnp.maximum(
      jnp.dot(cw2[...], r1, precision=hi,
              preferred_element_type=jnp.float32) + cb2[...], 0.0)
  rgb_full = jax.nn.sigmoid(
      jnp.dot(cw3[...], r2, precision=hi,
              preferred_element_type=jnp.float32) + cb3[...])
  xb = x_ref[...]
  xn = xb / 3.0
  m = ((jnp.abs(xn[0:1, :]) < 0.5) & (jnp.abs(xn[1:2, :]) < 0.5)
       & (jnp.abs(xn[2:3, :]) < 0.5))
  sig_ref[...] = jnp.exp(jnp.where(m, h[0:1, :], -100000.0))
  rgb_ref[...] = jnp.where(m, rgb_full, 0.0)


def _tc_mlp(featsT, dT, xT, w1, b1, w2, b2, cw1h, cw1d, cb1, cw2, cb2, cw3,
            cb3):
  grid = (_B // _BT,)

  def batch_spec(rows):
    return pl.BlockSpec((rows, _BT), lambda i: (0, i))

  def full_spec(shape):
    return pl.BlockSpec(shape, lambda i: (0, 0))

  return pl.pallas_call(
      _tc_mlp_body,
      grid=grid,
      in_specs=[
          batch_spec(2 * _NLEV),
          batch_spec(3),
          batch_spec(3),
          full_spec(w1.shape), full_spec(b1.shape),
          full_spec(w2.shape), full_spec(b2.shape),
          full_spec(cw1h.shape), full_spec(cw1d.shape), full_spec(cb1.shape),
          full_spec(cw2.shape), full_spec(cb2.shape),
          full_spec(cw3.shape), full_spec(cb3.shape),
      ],
      out_specs=[batch_spec(3), batch_spec(1)],
      out_shape=[
          jax.ShapeDtypeStruct((3, _B), jnp.float32),
          jax.ShapeDtypeStruct((1, _B), jnp.float32),
      ],
  )(featsT, dT, xT, w1, b1, w2, b2, cw1h, cw1d, cb1, cw2, cb2, cw3, cb3)


def kernel(x, d, tables, d_w1, d_b1, d_w2, d_b2, c_w1, c_b1, c_w2, c_b2,
           c_w3, c_b3):
  x = x.astype(jnp.float32)
  d = d.astype(jnp.float32)
  xT = x.T
  dT = d.T
  tblp = tables.astype(jnp.float32).reshape(_RPK, 16)

  featsT = _sc_hash_features(xT, tblp)

  rgbT, sigT = _tc_mlp(
      featsT, dT, xT,
      d_w1, d_b1.reshape(-1, 1),
      d_w2, d_b2.reshape(-1, 1),
      c_w1[:, :16], c_w1[:, 16:], c_b1.reshape(-1, 1),
      c_w2, c_b2.reshape(-1, 1),
      c_w3, c_b3.reshape(-1, 1),
  )
  return rgbT.T, sigT.reshape(_B)
